# Initial kernel scaffold; baseline (speedup 1.0000x reference)
import jax
import jax.numpy as jnp
from jax.experimental import pallas as pl


def kernel(X, edge_index, W, W_ih, W_hh, b_ih, b_hh):
    raise NotImplementedError("write your pallas kernel here")


# speedup vs baseline: 23.5730x; 23.5730x over previous
import jax
import jax.numpy as jnp
from jax import lax
from jax.experimental import pallas as pl
from jax.experimental.pallas import tpu as pltpu
from jax.experimental.pallas import tpu_sc as plsc

N = 10000
D = 128
DH = D // 2
E = 320000
NC = 2
NS = 16
B = 128
EPB = 160
EPS = EPB * B
EP = NS * EPS
HB = EPB // NC
NPAD = 10240
RPT = NPAD // NS
RB = 1024


def _fill_zeros_2d(ref, nrows, ncols):
    def body(i, c):
        for k in range(ncols // 16):
            ref[i, pl.ds(k * 16, 16)] = jnp.zeros((16,), jnp.float32)
        return c
    lax.fori_loop(0, nrows, body, 0)


def _deg_body(col_hbm, d0_hbm, d1_hbm, col_v, ones_v, zero_v, deg_sh):
    cid = lax.axis_index("c")
    sid = lax.axis_index("s")
    def zb(i, c):
        zero_v[pl.ds(i * 16, 16)] = jnp.zeros((16,), jnp.float32)
        return c
    lax.fori_loop(0, RPT // 16, zb, 0)
    pltpu.sync_copy(zero_v, deg_sh.at[pl.ds(sid * RPT, RPT)])
    def ob(i, c):
        ones_v[pl.ds(i * 16, 16)] = jnp.ones((16,), jnp.float32)
        return c
    lax.fori_loop(0, B // 16, ob, 0)
    plsc.subcore_barrier()
    pltpu.sync_copy(col_hbm.at[sid], col_v)
    def body(j, c):
        pltpu.sync_copy(ones_v, deg_sh.at[col_v.at[j]], add=True)
        return c
    lax.fori_loop(cid * HB, cid * HB + HB, body, 0)
    plsc.subcore_barrier()
    my_rows = deg_sh.at[pl.ds(sid * RPT, RPT)]
    @pl.when(cid == 0)
    def _():
        pltpu.sync_copy(my_rows, d0_hbm.at[pl.ds(sid * RPT, RPT)])
    @pl.when(cid == 1)
    def _():
        pltpu.sync_copy(my_rows, d1_hbm.at[pl.ds(sid * RPT, RPT)])


_deg_call = pl.kernel(
    _deg_body,
    out_type=(jax.ShapeDtypeStruct((NPAD,), jnp.float32),
              jax.ShapeDtypeStruct((NPAD,), jnp.float32)),
    mesh=plsc.VectorSubcoreMesh(core_axis_name="c", subcore_axis_name="s",
                                num_cores=NC, num_subcores=NS),
    scratch_types=[
        pltpu.VMEM((EPB, B), jnp.int32),
        pltpu.VMEM((B,), jnp.float32),
        pltpu.VMEM((RPT,), jnp.float32),
        pltpu.VMEM_SHARED((NPAD,), jnp.float32),
    ],
)


def _gs_body(y0_hbm, y1_hbm, row_hbm, col_hbm, s0_hbm, s1_hbm,
             row_v, col_v, rows_v, zrow_v, s_sh, gsem):
    cid = lax.axis_index("c")
    sid = lax.axis_index("s")
    _fill_zeros_2d(zrow_v, B, DH)
    def zb(t, c):
        pltpu.sync_copy(zrow_v, s_sh.at[pl.ds(sid * RPT + t * B, B)])
        return c
    lax.fori_loop(0, RPT // B, zb, 0)
    plsc.subcore_barrier()
    pltpu.sync_copy(row_hbm.at[sid], row_v)
    pltpu.sync_copy(col_hbm.at[sid], col_v)

    def mk(y_hbm):
        def body(j, c):
            pltpu.async_copy(y_hbm.at[row_v.at[j]], rows_v, gsem).wait()
            pltpu.sync_copy(rows_v, s_sh.at[col_v.at[j]], add=True)
            return c
        return body
    @pl.when(cid == 0)
    def _():
        lax.fori_loop(0, EPB, mk(y0_hbm), 0)
    @pl.when(cid == 1)
    def _():
        lax.fori_loop(0, EPB, mk(y1_hbm), 0)
    plsc.subcore_barrier()
    my_rows = s_sh.at[pl.ds(sid * RPT, RPT)]
    @pl.when(cid == 0)
    def _():
        pltpu.sync_copy(my_rows, s0_hbm.at[pl.ds(sid * RPT, RPT)])
    @pl.when(cid == 1)
    def _():
        pltpu.sync_copy(my_rows, s1_hbm.at[pl.ds(sid * RPT, RPT)])


_gs_call = pl.kernel(
    _gs_body,
    out_type=(jax.ShapeDtypeStruct((NPAD, DH), jnp.float32),
              jax.ShapeDtypeStruct((NPAD, DH), jnp.float32)),
    mesh=plsc.VectorSubcoreMesh(core_axis_name="c", subcore_axis_name="s",
                                num_cores=NC, num_subcores=NS),
    scratch_types=[
        pltpu.VMEM((EPB, B), jnp.int32),
        pltpu.VMEM((EPB, B), jnp.int32),
        pltpu.VMEM((B, DH), jnp.float32),
        pltpu.VMEM((B, DH), jnp.float32),
        pltpu.VMEM_SHARED((NPAD, DH), jnp.float32),
        pltpu.SemaphoreType.DMA,
    ],
    compiler_params=pltpu.CompilerParams(use_tc_tiling_on_sc=False),
)


def _sigmoid(x):
    return 1.0 / (1.0 + jnp.exp(-x))


def _lstm_body(w_ref, wihT_ref, bi_ref, bh_ref, out_ref):
    g = jnp.dot(w_ref[...], wihT_ref[...], preferred_element_type=jnp.float32)
    g = g + bi_ref[...] + bh_ref[...]
    i = g[:, :D]
    gg = g[:, 2 * D:3 * D]
    o = g[:, 3 * D:]
    c = _sigmoid(i) * jnp.tanh(gg)
    out_ref[...] = _sigmoid(o) * jnp.tanh(c)


def _lstm_call(W, W_ihT, bi, bh):
    return pl.pallas_call(
        _lstm_body,
        out_shape=jax.ShapeDtypeStruct((D, D), jnp.float32),
    )(W, W_ihT, bi, bh)


def _y_body(x_ref, w_ref, degp_ref, y_ref):
    dp = degp_ref[...]
    dis = lax.rsqrt(dp[0] + dp[1] + 1.0)
    y = jnp.dot(x_ref[...], w_ref[...], preferred_element_type=jnp.float32) * dis
    y_ref[0] = y[:, :DH]
    y_ref[1] = y[:, DH:]


def _y_call(X_pad, W_ev, degp3):
    return pl.pallas_call(
        _y_body,
        grid=(NPAD // RB,),
        in_specs=[
            pl.BlockSpec((RB, D), lambda i: (i, 0)),
            pl.BlockSpec((D, D), lambda i: (0, 0)),
            pl.BlockSpec((NC, RB, 1), lambda i: (0, i, 0)),
        ],
        out_specs=pl.BlockSpec((NC, RB, DH), lambda i: (0, i, 0)),
        out_shape=jax.ShapeDtypeStruct((NC, NPAD, DH), jnp.float32),
    )(X_pad, W_ev, degp3)


def _out_body(s_ref, y_ref, degp_ref, o_ref):
    dp = degp_ref[...]
    dis = lax.rsqrt(dp[0] + dp[1] + 1.0)
    s = s_ref[...]
    y = y_ref[...]
    o_ref[:, :DH] = (s[0] + y[0]) * dis
    o_ref[:, DH:] = (s[1] + y[1]) * dis


def _out_call(S2, Y2, degp3):
    return pl.pallas_call(
        _out_body,
        grid=(NPAD // RB,),
        in_specs=[
            pl.BlockSpec((NC, RB, DH), lambda i: (0, i, 0)),
            pl.BlockSpec((NC, RB, DH), lambda i: (0, i, 0)),
            pl.BlockSpec((NC, RB, 1), lambda i: (0, i, 0)),
        ],
        out_specs=pl.BlockSpec((RB, D), lambda i: (i, 0)),
        out_shape=jax.ShapeDtypeStruct((NPAD, D), jnp.float32),
    )(S2, Y2, degp3)


def kernel(X, edge_index, W, W_ih, W_hh, b_ih, b_hh):
    row = edge_index[0]
    col = edge_index[1]
    P = EP - E
    pad_idx = (jnp.arange(P, dtype=jnp.int32) % (NPAD - N)) + N
    row_p = jnp.concatenate([row, pad_idx]).reshape(NS, EPB, B)
    col_p = jnp.concatenate([col, pad_idx]).reshape(NS, EPB, B)
    X_pad = jnp.pad(X, ((0, NPAD - N), (0, 0)))

    d0, d1 = _deg_call(col_p)
    degp3 = jnp.stack([d0, d1]).reshape(NC, NPAD, 1)
    W_ev = _lstm_call(W, W_ih.T, b_ih.reshape(1, 4 * D), b_hh.reshape(1, 4 * D))
    Y2 = _y_call(X_pad, W_ev, degp3)
    s0, s1 = _gs_call(Y2[0], Y2[1], row_p, col_p)
    S2 = jnp.stack([s0, s1])
    out = _out_call(S2, Y2, degp3)
    return out[:N]


# speedup vs baseline: 42.8666x; 1.8185x over previous
import jax
import jax.numpy as jnp
from jax import lax
from jax.experimental import pallas as pl
from jax.experimental.pallas import tpu as pltpu
from jax.experimental.pallas import tpu_sc as plsc

N = 10000
D = 128
DH = D // 2
E = 320000
NC = 2
NS = 16
B = 128
EPB = 160
EPS = EPB * B
EP = NS * EPS
HB = EPB // NC
NPAD = 10240
RPT = NPAD // NS
RB = 1000
NBUF = 4


def _deg_body(col_hbm, d0_hbm, d1_hbm, col_v, ones_v, zero_v, deg_sh):
    cid = lax.axis_index("c")
    sid = lax.axis_index("s")
    def zb(i, c):
        zero_v[pl.ds(i * 16, 16)] = jnp.zeros((16,), jnp.float32)
        return c
    lax.fori_loop(0, RPT // 16, zb, 0)
    pltpu.sync_copy(zero_v, deg_sh.at[pl.ds(sid * RPT, RPT)])
    def ob(i, c):
        ones_v[pl.ds(i * 16, 16)] = jnp.ones((16,), jnp.float32)
        return c
    lax.fori_loop(0, B // 16, ob, 0)
    plsc.subcore_barrier()
    pltpu.sync_copy(col_hbm.at[sid], col_v)
    def body(j, c):
        pltpu.sync_copy(ones_v, deg_sh.at[col_v.at[j]], add=True)
        return c
    lax.fori_loop(cid * HB, cid * HB + HB, body, 0)
    plsc.subcore_barrier()
    my_rows = deg_sh.at[pl.ds(sid * RPT, RPT)]
    @pl.when(cid == 0)
    def _():
        pltpu.sync_copy(my_rows, d0_hbm.at[pl.ds(sid * RPT, RPT)])
    @pl.when(cid == 1)
    def _():
        pltpu.sync_copy(my_rows, d1_hbm.at[pl.ds(sid * RPT, RPT)])


_deg_call = pl.kernel(
    _deg_body,
    out_type=(jax.ShapeDtypeStruct((NPAD,), jnp.float32),
              jax.ShapeDtypeStruct((NPAD,), jnp.float32)),
    mesh=plsc.VectorSubcoreMesh(core_axis_name="c", subcore_axis_name="s",
                                num_cores=NC, num_subcores=NS),
    scratch_types=[
        pltpu.VMEM((EPB, B), jnp.int32),
        pltpu.VMEM((B,), jnp.float32),
        pltpu.VMEM((RPT,), jnp.float32),
        pltpu.VMEM_SHARED((NPAD,), jnp.float32),
    ],
)


def _gs_body(y0_hbm, y1_hbm, row_hbm, col_hbm, s0_hbm, s1_hbm,
             row_v, col_v, b0, b1, b2, b3, zrow_v, s_sh, g0, g1, g2, g3):
    cid = lax.axis_index("c")
    sid = lax.axis_index("s")
    bufs = (b0, b1, b2, b3)
    sems = (g0, g1, g2, g3)
    def fz(i, c):
        for k in range(DH // 16):
            zrow_v[i, pl.ds(k * 16, 16)] = jnp.zeros((16,), jnp.float32)
        return c
    lax.fori_loop(0, B, fz, 0)
    def zb(t, c):
        pltpu.sync_copy(zrow_v, s_sh.at[pl.ds(sid * RPT + t * B, B)])
        return c
    lax.fori_loop(0, RPT // B, zb, 0)
    plsc.subcore_barrier()
    pltpu.sync_copy(row_hbm.at[sid], row_v)
    pltpu.sync_copy(col_hbm.at[sid], col_v)

    def run(y_hbm):
        for b in range(NBUF):
            pltpu.async_copy(y_hbm.at[row_v.at[b]], bufs[b], sems[b])
        def outer(t, c):
            for b in range(NBUF):
                j = t * NBUF + b
                pltpu.make_async_copy(y_hbm.at[row_v.at[j]], bufs[b],
                                      sems[b]).wait()
                pltpu.sync_copy(bufs[b], s_sh.at[col_v.at[j]], add=True)
                @pl.when(j + NBUF < EPB)
                def _():
                    pltpu.async_copy(y_hbm.at[row_v.at[j + NBUF]], bufs[b],
                                     sems[b])
            return c
        lax.fori_loop(0, EPB // NBUF, outer, 0)
    @pl.when(cid == 0)
    def _():
        run(y0_hbm)
    @pl.when(cid == 1)
    def _():
        run(y1_hbm)
    plsc.subcore_barrier()
    my_rows = s_sh.at[pl.ds(sid * RPT, RPT)]
    @pl.when(cid == 0)
    def _():
        pltpu.sync_copy(my_rows, s0_hbm.at[pl.ds(sid * RPT, RPT)])
    @pl.when(cid == 1)
    def _():
        pltpu.sync_copy(my_rows, s1_hbm.at[pl.ds(sid * RPT, RPT)])


_gs_call = pl.kernel(
    _gs_body,
    out_type=(jax.ShapeDtypeStruct((NPAD, DH), jnp.float32),
              jax.ShapeDtypeStruct((NPAD, DH), jnp.float32)),
    mesh=plsc.VectorSubcoreMesh(core_axis_name="c", subcore_axis_name="s",
                                num_cores=NC, num_subcores=NS),
    scratch_types=[
        pltpu.VMEM((EPB, B), jnp.int32),
        pltpu.VMEM((EPB, B), jnp.int32),
        pltpu.VMEM((B, DH), jnp.float32),
        pltpu.VMEM((B, DH), jnp.float32),
        pltpu.VMEM((B, DH), jnp.float32),
        pltpu.VMEM((B, DH), jnp.float32),
        pltpu.VMEM((B, DH), jnp.float32),
        pltpu.VMEM_SHARED((NPAD, DH), jnp.float32),
        pltpu.SemaphoreType.DMA,
        pltpu.SemaphoreType.DMA,
        pltpu.SemaphoreType.DMA,
        pltpu.SemaphoreType.DMA,
    ],
    compiler_params=pltpu.CompilerParams(use_tc_tiling_on_sc=False),
)


def _sigmoid(x):
    return 1.0 / (1.0 + jnp.exp(-x))


def _lstm_body(w_ref, wihT_ref, bi_ref, bh_ref, out_ref):
    g = jnp.dot(w_ref[...], wihT_ref[...], preferred_element_type=jnp.float32)
    g = g + bi_ref[...] + bh_ref[...]
    i = g[:, :D]
    gg = g[:, 2 * D:3 * D]
    o = g[:, 3 * D:]
    c = _sigmoid(i) * jnp.tanh(gg)
    out_ref[...] = _sigmoid(o) * jnp.tanh(c)


def _lstm_call(W, W_ihT, bi, bh):
    return pl.pallas_call(
        _lstm_body,
        out_shape=jax.ShapeDtypeStruct((D, D), jnp.float32),
    )(W, W_ihT, bi, bh)


def _dis(d0_ref, d1_ref):
    return lax.rsqrt(d0_ref[...] + d1_ref[...] + 1.0)


def _y_body(x_ref, w_ref, d0_ref, d1_ref, y0_ref, y1_ref):
    y = jnp.dot(x_ref[...], w_ref[...],
                preferred_element_type=jnp.float32) * _dis(d0_ref, d1_ref)
    y0_ref[...] = y[:, :DH]
    y1_ref[...] = y[:, DH:]


def _y_call(X, W_ev, d0, d1):
    return pl.pallas_call(
        _y_body,
        grid=(N // RB,),
        in_specs=[
            pl.BlockSpec((RB, D), lambda i: (i, 0)),
            pl.BlockSpec((D, D), lambda i: (0, 0)),
            pl.BlockSpec((RB, 1), lambda i: (i, 0)),
            pl.BlockSpec((RB, 1), lambda i: (i, 0)),
        ],
        out_specs=(pl.BlockSpec((RB, DH), lambda i: (i, 0)),
                   pl.BlockSpec((RB, DH), lambda i: (i, 0))),
        out_shape=(jax.ShapeDtypeStruct((N, DH), jnp.float32),
                   jax.ShapeDtypeStruct((N, DH), jnp.float32)),
    )(X, W_ev, d0, d1)


def _out_body(s0_ref, s1_ref, y0_ref, y1_ref, d0_ref, d1_ref, o_ref):
    dis = _dis(d0_ref, d1_ref)
    o_ref[:, :DH] = (s0_ref[...] + y0_ref[...]) * dis
    o_ref[:, DH:] = (s1_ref[...] + y1_ref[...]) * dis


def _out_call(s0, s1, y0, y1, d0, d1):
    return pl.pallas_call(
        _out_body,
        grid=(N // RB,),
        in_specs=[
            pl.BlockSpec((RB, DH), lambda i: (i, 0)),
            pl.BlockSpec((RB, DH), lambda i: (i, 0)),
            pl.BlockSpec((RB, DH), lambda i: (i, 0)),
            pl.BlockSpec((RB, DH), lambda i: (i, 0)),
            pl.BlockSpec((RB, 1), lambda i: (i, 0)),
            pl.BlockSpec((RB, 1), lambda i: (i, 0)),
        ],
        out_specs=pl.BlockSpec((RB, D), lambda i: (i, 0)),
        out_shape=jax.ShapeDtypeStruct((N, D), jnp.float32),
    )(s0, s1, y0, y1, d0, d1)


def kernel(X, edge_index, W, W_ih, W_hh, b_ih, b_hh):
    row = edge_index[0]
    col = edge_index[1]
    P = EP - E
    pad_gather = jnp.arange(P, dtype=jnp.int32) % N
    pad_scatter = (jnp.arange(P, dtype=jnp.int32) % (NPAD - N)) + N
    row_p = jnp.concatenate([row, pad_gather]).reshape(NS, EPB, B)
    col_p = jnp.concatenate([col, pad_scatter]).reshape(NS, EPB, B)

    d0, d1 = _deg_call(col_p)
    d0 = d0.reshape(NPAD, 1)
    d1 = d1.reshape(NPAD, 1)
    W_ev = _lstm_call(W, W_ih.T, b_ih.reshape(1, 4 * D), b_hh.reshape(1, 4 * D))
    y0, y1 = _y_call(X, W_ev, d0, d1)
    s0, s1 = _gs_call(y0, y1, row_p, col_p)
    return _out_call(s0, s1, y0, y1, d0, d1)


# speedup vs baseline: 47.5456x; 1.1092x over previous
import jax
import jax.numpy as jnp
from jax import lax
from jax.experimental import pallas as pl
from jax.experimental.pallas import tpu as pltpu
from jax.experimental.pallas import tpu_sc as plsc

N = 10000
D = 128
DH = D // 2
E = 320000
NC = 2
NS = 16
B = 128
EPB = 160
EPS = EPB * B
EP = NS * EPS
HB = EPB // NC
NPAD = 10240
RPT = NPAD // NS
RB = 1000
NBUF = 4

_UNTILED = pltpu.CompilerParams(use_tc_tiling_on_sc=False)


def _deg_body(col_hbm, d0_hbm, d1_hbm, col_v, ones_v, zero_v, deg_sh):
    cid = lax.axis_index("c")
    sid = lax.axis_index("s")
    def zb(i, c):
        zero_v[pl.ds(i * 16, 16)] = jnp.zeros((16,), jnp.float32)
        return c
    lax.fori_loop(0, RPT // 16, zb, 0)
    pltpu.sync_copy(zero_v, deg_sh.at[pl.ds(sid * RPT, RPT)])
    def ob(i, c):
        ones_v[pl.ds(i * 16, 16)] = jnp.ones((16,), jnp.float32)
        return c
    lax.fori_loop(0, B // 16, ob, 0)
    plsc.subcore_barrier()
    pltpu.sync_copy(col_hbm.at[sid], col_v)
    def body(j, c):
        pltpu.sync_copy(ones_v, deg_sh.at[col_v.at[j]], add=True)
        return c
    lax.fori_loop(cid * HB, cid * HB + HB, body, 0)
    plsc.subcore_barrier()
    my_rows = deg_sh.at[pl.ds(sid * RPT, RPT)]
    @pl.when(cid == 0)
    def _():
        pltpu.sync_copy(my_rows, d0_hbm.at[pl.ds(sid * RPT, RPT)])
    @pl.when(cid == 1)
    def _():
        pltpu.sync_copy(my_rows, d1_hbm.at[pl.ds(sid * RPT, RPT)])


_deg_call = pl.kernel(
    _deg_body,
    out_type=(jax.ShapeDtypeStruct((NPAD,), jnp.float32),
              jax.ShapeDtypeStruct((NPAD,), jnp.float32)),
    mesh=plsc.VectorSubcoreMesh(core_axis_name="c", subcore_axis_name="s",
                                num_cores=NC, num_subcores=NS),
    scratch_types=[
        pltpu.VMEM((EPB, B), jnp.int32),
        pltpu.VMEM((B,), jnp.float32),
        pltpu.VMEM((RPT,), jnp.float32),
        pltpu.VMEM_SHARED((NPAD,), jnp.float32),
    ],
    compiler_params=_UNTILED,
)


def _gs_body(y_hbm, row_hbm, col_hbm, s_hbm,
             row_v, col_v, b0, b1, b2, b3, zrow_v, s_sh, g0, g1, g2, g3):
    cid = lax.axis_index("c")
    sid = lax.axis_index("s")
    bufs = (b0, b1, b2, b3)
    sems = (g0, g1, g2, g3)
    def fz(i, c):
        for k in range(DH // 16):
            zrow_v[i, pl.ds(k * 16, 16)] = jnp.zeros((16,), jnp.float32)
        return c
    lax.fori_loop(0, B, fz, 0)
    def zb(t, c):
        pltpu.sync_copy(zrow_v, s_sh.at[pl.ds(sid * RPT + t * B, B)])
        return c
    lax.fori_loop(0, RPT // B, zb, 0)
    plsc.subcore_barrier()
    pltpu.sync_copy(row_hbm.at[sid], row_v)
    pltpu.sync_copy(col_hbm.at[sid], col_v)
    cvec = jnp.full((16,), 0, jnp.int32) + cid
    def rm(j, c):
        for k in range(B // 16):
            v = row_v[j, pl.ds(k * 16, 16)]
            row_v[j, pl.ds(k * 16, 16)] = v + v + cvec
        return c
    lax.fori_loop(0, EPB, rm, 0)

    for b in range(NBUF):
        pltpu.async_copy(y_hbm.at[row_v.at[b]], bufs[b], sems[b])
    def outer(t, c):
        for b in range(NBUF):
            j = t * NBUF + b
            pltpu.make_async_copy(y_hbm.at[row_v.at[j]], bufs[b],
                                  sems[b]).wait()
            pltpu.sync_copy(bufs[b], s_sh.at[col_v.at[j]], add=True)
            @pl.when(j + NBUF < EPB)
            def _():
                pltpu.async_copy(y_hbm.at[row_v.at[j + NBUF]], bufs[b],
                                 sems[b])
        return c
    lax.fori_loop(0, EPB // NBUF, outer, 0)
    plsc.subcore_barrier()
    pltpu.sync_copy(s_sh.at[pl.ds(sid * RPT, RPT)],
                    s_hbm.at[pl.ds(sid * RPT, RPT), pl.ds(cid * DH, DH)])


_gs_call = pl.kernel(
    _gs_body,
    out_type=jax.ShapeDtypeStruct((NPAD, D), jnp.float32),
    mesh=plsc.VectorSubcoreMesh(core_axis_name="c", subcore_axis_name="s",
                                num_cores=NC, num_subcores=NS),
    scratch_types=[
        pltpu.VMEM((EPB, B), jnp.int32),
        pltpu.VMEM((EPB, B), jnp.int32),
        pltpu.VMEM((B, DH), jnp.float32),
        pltpu.VMEM((B, DH), jnp.float32),
        pltpu.VMEM((B, DH), jnp.float32),
        pltpu.VMEM((B, DH), jnp.float32),
        pltpu.VMEM((B, DH), jnp.float32),
        pltpu.VMEM_SHARED((NPAD, DH), jnp.float32),
        pltpu.SemaphoreType.DMA,
        pltpu.SemaphoreType.DMA,
        pltpu.SemaphoreType.DMA,
        pltpu.SemaphoreType.DMA,
    ],
    compiler_params=_UNTILED,
)


def _sigmoid(x):
    return 1.0 / (1.0 + jnp.exp(-x))


def _dis(d0_ref, d1_ref):
    return lax.rsqrt(d0_ref[...] + d1_ref[...] + 1.0)


def _y_body(x_ref, w_ref, wihT_ref, bi_ref, bh_ref, d0_ref, d1_ref,
            y_ref, wev_ref):
    @pl.when(pl.program_id(0) == 0)
    def _():
        g = jnp.dot(w_ref[...], wihT_ref[...],
                    preferred_element_type=jnp.float32)
        g = g + bi_ref[...] + bh_ref[...]
        i = g[:, :D]
        gg = g[:, 2 * D:3 * D]
        o = g[:, 3 * D:]
        c = _sigmoid(i) * jnp.tanh(gg)
        wev_ref[...] = _sigmoid(o) * jnp.tanh(c)
    y_ref[...] = jnp.dot(x_ref[...], wev_ref[...],
                         preferred_element_type=jnp.float32) * _dis(d0_ref,
                                                                    d1_ref)


def _y_call(X, W, W_ihT, bi, bh, d0, d1):
    return pl.pallas_call(
        _y_body,
        grid=(N // RB,),
        in_specs=[
            pl.BlockSpec((RB, D), lambda i: (i, 0)),
            pl.BlockSpec((D, D), lambda i: (0, 0)),
            pl.BlockSpec((D, 4 * D), lambda i: (0, 0)),
            pl.BlockSpec((1, 4 * D), lambda i: (0, 0)),
            pl.BlockSpec((1, 4 * D), lambda i: (0, 0)),
            pl.BlockSpec((RB, 1), lambda i: (i, 0)),
            pl.BlockSpec((RB, 1), lambda i: (i, 0)),
        ],
        out_specs=pl.BlockSpec((RB, D), lambda i: (i, 0)),
        out_shape=jax.ShapeDtypeStruct((N, D), jnp.float32),
        scratch_shapes=[pltpu.VMEM((D, D), jnp.float32)],
    )(X, W, W_ihT, bi, bh, d0, d1)


def _out_body(s_ref, y_ref, d0_ref, d1_ref, o_ref):
    o_ref[...] = (s_ref[...] + y_ref[...]) * _dis(d0_ref, d1_ref)


def _out_call(s, y, d0, d1):
    return pl.pallas_call(
        _out_body,
        grid=(N // RB,),
        in_specs=[
            pl.BlockSpec((RB, D), lambda i: (i, 0)),
            pl.BlockSpec((RB, D), lambda i: (i, 0)),
            pl.BlockSpec((RB, 1), lambda i: (i, 0)),
            pl.BlockSpec((RB, 1), lambda i: (i, 0)),
        ],
        out_specs=pl.BlockSpec((RB, D), lambda i: (i, 0)),
        out_shape=jax.ShapeDtypeStruct((N, D), jnp.float32),
    )(s, y, d0, d1)


def kernel(X, edge_index, W, W_ih, W_hh, b_ih, b_hh):
    row = edge_index[0]
    col = edge_index[1]
    P = EP - E
    pad_gather = jnp.arange(P, dtype=jnp.int32) % N
    pad_scatter = (jnp.arange(P, dtype=jnp.int32) % (NPAD - N)) + N
    row_p = jnp.concatenate([row, pad_gather]).reshape(NS, EPB, B)
    col_p = jnp.concatenate([col, pad_scatter]).reshape(NS, EPB, B)

    d0, d1 = _deg_call(col_p)
    d0 = d0.reshape(NPAD, 1)
    d1 = d1.reshape(NPAD, 1)
    Y = _y_call(X, W, W_ih.T, b_ih.reshape(1, 4 * D), b_hh.reshape(1, 4 * D),
                d0, d1)
    y_view = Y.reshape(2 * N, DH)
    S = _gs_call(y_view, row_p, col_p)
    return _out_call(S, Y, d0, d1)


# speedup vs baseline: 49.2308x; 1.0354x over previous
import jax
import jax.numpy as jnp
from jax import lax
from jax.experimental import pallas as pl
from jax.experimental.pallas import tpu as pltpu
from jax.experimental.pallas import tpu_sc as plsc

N = 10000
D = 128
DH = D // 2
E = 320000
NC = 2
NS = 16
B = 128
NBT = E // B
BPT = NBT // NS
XTRA = NBT - BPT * NS
MAXB = BPT + 1
NPAD = 10240
RPT = NPAD // NS
RB = 1000
NBUF = 4

_UNTILED = pltpu.CompilerParams(use_tc_tiling_on_sc=False)


def _tile_range(sid):
    start = BPT * sid + jnp.minimum(sid, XTRA)
    nb = BPT + (sid < XTRA).astype(jnp.int32)
    return start, nb


def _deg_body(col_hbm, d0_hbm, d1_hbm, col_v, ones_v, zero_v, deg_sh):
    cid = lax.axis_index("c")
    sid = lax.axis_index("s")
    start, nb = _tile_range(sid)
    def zb(i, c):
        zero_v[pl.ds(i * 16, 16)] = jnp.zeros((16,), jnp.float32)
        return c
    lax.fori_loop(0, RPT // 16, zb, 0)
    pltpu.sync_copy(zero_v, deg_sh.at[pl.ds(sid * RPT, RPT)])
    def ob(i, c):
        ones_v[pl.ds(i * 16, 16)] = jnp.ones((16,), jnp.float32)
        return c
    lax.fori_loop(0, B // 16, ob, 0)
    plsc.subcore_barrier()
    @pl.when(sid < XTRA)
    def _():
        pltpu.sync_copy(col_hbm.at[pl.ds(start, MAXB)], col_v)
    @pl.when(sid >= XTRA)
    def _():
        pltpu.sync_copy(col_hbm.at[pl.ds(start, BPT)], col_v.at[pl.ds(0, BPT)])
    half = (nb + 1) // 2
    j0 = cid * half
    j1 = j0 + jnp.where(cid == 0, half, nb - half)
    def body(j, c):
        pltpu.sync_copy(ones_v, deg_sh.at[col_v.at[j]], add=True)
        return c
    lax.fori_loop(j0, j1, body, 0)
    plsc.subcore_barrier()
    my_rows = deg_sh.at[pl.ds(sid * RPT, RPT)]
    @pl.when(cid == 0)
    def _():
        pltpu.sync_copy(my_rows, d0_hbm.at[pl.ds(sid * RPT, RPT)])
    @pl.when(cid == 1)
    def _():
        pltpu.sync_copy(my_rows, d1_hbm.at[pl.ds(sid * RPT, RPT)])


_deg_call = pl.kernel(
    _deg_body,
    out_type=(jax.ShapeDtypeStruct((NPAD,), jnp.float32),
              jax.ShapeDtypeStruct((NPAD,), jnp.float32)),
    mesh=plsc.VectorSubcoreMesh(core_axis_name="c", subcore_axis_name="s",
                                num_cores=NC, num_subcores=NS),
    scratch_types=[
        pltpu.VMEM((MAXB, B), jnp.int32),
        pltpu.VMEM((B,), jnp.float32),
        pltpu.VMEM((RPT,), jnp.float32),
        pltpu.VMEM_SHARED((NPAD,), jnp.float32),
    ],
    compiler_params=_UNTILED,
)


def _gs_body(y_hbm, row_hbm, col_hbm, s_hbm,
             row_v, col_v, b0, b1, b2, b3, zrow_v, s_sh, g0, g1, g2, g3):
    cid = lax.axis_index("c")
    sid = lax.axis_index("s")
    bufs = (b0, b1, b2, b3)
    sems = (g0, g1, g2, g3)
    start, nb = _tile_range(sid)
    def fz(i, c):
        for k in range(DH // 16):
            zrow_v[i, pl.ds(k * 16, 16)] = jnp.zeros((16,), jnp.float32)
        return c
    lax.fori_loop(0, B, fz, 0)
    def zb(t, c):
        pltpu.sync_copy(zrow_v, s_sh.at[pl.ds(sid * RPT + t * B, B)])
        return c
    lax.fori_loop(0, RPT // B, zb, 0)
    plsc.subcore_barrier()
    @pl.when(sid < XTRA)
    def _():
        pltpu.sync_copy(row_hbm.at[pl.ds(start, MAXB)], row_v)
        pltpu.sync_copy(col_hbm.at[pl.ds(start, MAXB)], col_v)
    @pl.when(sid >= XTRA)
    def _():
        pltpu.sync_copy(row_hbm.at[pl.ds(start, BPT)], row_v.at[pl.ds(0, BPT)])
        pltpu.sync_copy(col_hbm.at[pl.ds(start, BPT)], col_v.at[pl.ds(0, BPT)])
    cvec = jnp.full((16,), 0, jnp.int32) + cid
    def rm(j, c):
        for k in range(B // 16):
            v = row_v[j, pl.ds(k * 16, 16)]
            row_v[j, pl.ds(k * 16, 16)] = v + v + cvec
        return c
    lax.fori_loop(0, nb, rm, 0)

    for b in range(NBUF):
        pltpu.async_copy(y_hbm.at[row_v.at[b]], bufs[b], sems[b])
    def outer(t, c):
        for b in range(NBUF):
            j = t * NBUF + b
            pltpu.make_async_copy(y_hbm.at[row_v.at[j]], bufs[b],
                                  sems[b]).wait()
            pltpu.sync_copy(bufs[b], s_sh.at[col_v.at[j]], add=True)
            @pl.when(j + NBUF < nb)
            def _():
                pltpu.async_copy(y_hbm.at[row_v.at[j + NBUF]], bufs[b],
                                 sems[b])
        return c
    lax.fori_loop(0, BPT // NBUF, outer, 0)
    @pl.when(sid < XTRA)
    def _():
        pltpu.make_async_copy(y_hbm.at[row_v.at[BPT]], bufs[0], sems[0]).wait()
        pltpu.sync_copy(bufs[0], s_sh.at[col_v.at[BPT]], add=True)
    plsc.subcore_barrier()
    pltpu.sync_copy(s_sh.at[pl.ds(sid * RPT, RPT)],
                    s_hbm.at[pl.ds(sid * RPT, RPT), pl.ds(cid * DH, DH)])


_gs_call = pl.kernel(
    _gs_body,
    out_type=jax.ShapeDtypeStruct((NPAD, D), jnp.float32),
    mesh=plsc.VectorSubcoreMesh(core_axis_name="c", subcore_axis_name="s",
                                num_cores=NC, num_subcores=NS),
    scratch_types=[
        pltpu.VMEM((MAXB, B), jnp.int32),
        pltpu.VMEM((MAXB, B), jnp.int32),
        pltpu.VMEM((B, DH), jnp.float32),
        pltpu.VMEM((B, DH), jnp.float32),
        pltpu.VMEM((B, DH), jnp.float32),
        pltpu.VMEM((B, DH), jnp.float32),
        pltpu.VMEM((B, DH), jnp.float32),
        pltpu.VMEM_SHARED((NPAD, DH), jnp.float32),
        pltpu.SemaphoreType.DMA,
        pltpu.SemaphoreType.DMA,
        pltpu.SemaphoreType.DMA,
        pltpu.SemaphoreType.DMA,
    ],
    compiler_params=_UNTILED,
)


def _sigmoid(x):
    return 1.0 / (1.0 + jnp.exp(-x))


def _y_body(x_ref, w_ref, wihT_ref, bi_ref, bh_ref, dd_ref, y_ref, wev_ref):
    @pl.when(pl.program_id(0) == 0)
    def _():
        g = jnp.dot(w_ref[...], wihT_ref[...],
                    preferred_element_type=jnp.float32)
        g = g + bi_ref[...] + bh_ref[...]
        i = g[:, :D]
        gg = g[:, 2 * D:3 * D]
        o = g[:, 3 * D:]
        c = _sigmoid(i) * jnp.tanh(gg)
        wev_ref[...] = _sigmoid(o) * jnp.tanh(c)
    dis = lax.rsqrt(dd_ref[...] + 1.0)
    y_ref[...] = jnp.dot(x_ref[...], wev_ref[...],
                         preferred_element_type=jnp.float32) * dis


def _y_call(X, W, W_ihT, bi, bh, dd):
    return pl.pallas_call(
        _y_body,
        grid=(N // RB,),
        in_specs=[
            pl.BlockSpec((RB, D), lambda i: (i, 0)),
            pl.BlockSpec((D, D), lambda i: (0, 0)),
            pl.BlockSpec((D, 4 * D), lambda i: (0, 0)),
            pl.BlockSpec((1, 4 * D), lambda i: (0, 0)),
            pl.BlockSpec((1, 4 * D), lambda i: (0, 0)),
            pl.BlockSpec((RB, 1), lambda i: (i, 0)),
        ],
        out_specs=pl.BlockSpec((RB, D), lambda i: (i, 0)),
        out_shape=jax.ShapeDtypeStruct((N, D), jnp.float32),
        scratch_shapes=[pltpu.VMEM((D, D), jnp.float32)],
    )(X, W, W_ihT, bi, bh, dd)


def _out_body(s_ref, y_ref, dd_ref, o_ref):
    dis = lax.rsqrt(dd_ref[...] + 1.0)
    o_ref[...] = (s_ref[...] + y_ref[...]) * dis


def _out_call(s, y, dd):
    return pl.pallas_call(
        _out_body,
        grid=(N // RB,),
        in_specs=[
            pl.BlockSpec((RB, D), lambda i: (i, 0)),
            pl.BlockSpec((RB, D), lambda i: (i, 0)),
            pl.BlockSpec((RB, 1), lambda i: (i, 0)),
        ],
        out_specs=pl.BlockSpec((RB, D), lambda i: (i, 0)),
        out_shape=jax.ShapeDtypeStruct((N, D), jnp.float32),
    )(s, y, dd)


def kernel(X, edge_index, W, W_ih, W_hh, b_ih, b_hh):
    row2d = edge_index[0].reshape(NBT, B)
    col2d = edge_index[1].reshape(NBT, B)

    d0, d1 = _deg_call(col2d)
    dd = (d0 + d1).reshape(NPAD, 1)
    Y = _y_call(X, W, W_ih.T, b_ih.reshape(1, 4 * D), b_hh.reshape(1, 4 * D),
                dd)
    y_view = Y.reshape(2 * N, DH)
    S = _gs_call(y_view, row2d, col2d)
    return _out_call(S, Y, dd)


# speedup vs baseline: 54.0226x; 1.0973x over previous
import jax
import jax.numpy as jnp
from jax import lax
from jax.experimental import pallas as pl
from jax.experimental.pallas import tpu as pltpu
from jax.experimental.pallas import tpu_sc as plsc

N = 10000
D = 128
DH = D // 2
E = 320000
NC = 2
NS = 16
B = 128
NBT = E // B
BPT = NBT // NS
XTRA = NBT - BPT * NS
MAXB = BPT + 1
NPAD = 10240
RPT = NPAD // NS
RB = 1000
NBUF = 4

_UNTILED = pltpu.CompilerParams(use_tc_tiling_on_sc=False)


def _tile_range(sid):
    start = BPT * sid + jnp.minimum(sid, XTRA)
    nb = BPT + (sid < XTRA).astype(jnp.int32)
    return start, nb


def _deg_body(ei_hbm, d0_hbm, d1_hbm, col_v, ones_v, zero_v, deg_sh):
    cid = lax.axis_index("c")
    sid = lax.axis_index("s")
    start, nb = _tile_range(sid)
    def zb(i, c):
        zero_v[pl.ds(i * 16, 16)] = jnp.zeros((16,), jnp.float32)
        return c
    lax.fori_loop(0, RPT // 16, zb, 0)
    pltpu.sync_copy(zero_v, deg_sh.at[pl.ds(sid * RPT, RPT)])
    def ob(i, c):
        ones_v[pl.ds(i * 16, 16)] = jnp.ones((16,), jnp.float32)
        return c
    lax.fori_loop(0, B // 16, ob, 0)
    plsc.subcore_barrier()
    @pl.when(sid < XTRA)
    def _():
        pltpu.sync_copy(ei_hbm.at[pl.ds(start, MAXB), pl.ds(1, 1)], col_v)
    @pl.when(sid >= XTRA)
    def _():
        pltpu.sync_copy(ei_hbm.at[pl.ds(start, BPT), pl.ds(1, 1)],
                        col_v.at[pl.ds(0, BPT)])
    half = (nb + 1) // 2
    j0 = cid * half
    j1 = j0 + jnp.where(cid == 0, half, nb - half)
    def body(j, c):
        pltpu.sync_copy(ones_v, deg_sh.at[col_v.at[j, 0]], add=True)
        return c
    lax.fori_loop(j0, j1, body, 0)
    plsc.subcore_barrier()
    my_rows = deg_sh.at[pl.ds(sid * RPT, RPT)]
    @pl.when(cid == 0)
    def _():
        pltpu.sync_copy(my_rows, d0_hbm.at[pl.ds(sid * RPT, RPT)])
    @pl.when(cid == 1)
    def _():
        pltpu.sync_copy(my_rows, d1_hbm.at[pl.ds(sid * RPT, RPT)])


_deg_call = pl.kernel(
    _deg_body,
    out_type=(jax.ShapeDtypeStruct((NPAD,), jnp.float32),
              jax.ShapeDtypeStruct((NPAD,), jnp.float32)),
    mesh=plsc.VectorSubcoreMesh(core_axis_name="c", subcore_axis_name="s",
                                num_cores=NC, num_subcores=NS),
    scratch_types=[
        pltpu.VMEM((MAXB, 1, B), jnp.int32),
        pltpu.VMEM((B,), jnp.float32),
        pltpu.VMEM((RPT,), jnp.float32),
        pltpu.VMEM_SHARED((NPAD,), jnp.float32),
    ],
    compiler_params=_UNTILED,
)


def _gs_body(y_hbm, ei_hbm, s_hbm,
             row_v, col_v, b0, b1, b2, b3, zrow_v, s_sh, g0, g1, g2, g3):
    cid = lax.axis_index("c")
    sid = lax.axis_index("s")
    bufs = (b0, b1, b2, b3)
    sems = (g0, g1, g2, g3)
    start, nb = _tile_range(sid)
    def fz(i, c):
        for k in range(DH // 16):
            zrow_v[i, pl.ds(k * 16, 16)] = jnp.zeros((16,), jnp.float32)
        return c
    lax.fori_loop(0, B, fz, 0)
    def zb(t, c):
        pltpu.sync_copy(zrow_v, s_sh.at[pl.ds(sid * RPT + t * B, B)])
        return c
    lax.fori_loop(0, RPT // B, zb, 0)
    plsc.subcore_barrier()
    @pl.when(sid < XTRA)
    def _():
        pltpu.sync_copy(ei_hbm.at[pl.ds(start, MAXB), pl.ds(0, 1)], row_v)
        pltpu.sync_copy(ei_hbm.at[pl.ds(start, MAXB), pl.ds(1, 1)], col_v)
    @pl.when(sid >= XTRA)
    def _():
        pltpu.sync_copy(ei_hbm.at[pl.ds(start, BPT), pl.ds(0, 1)],
                        row_v.at[pl.ds(0, BPT)])
        pltpu.sync_copy(ei_hbm.at[pl.ds(start, BPT), pl.ds(1, 1)],
                        col_v.at[pl.ds(0, BPT)])
    cvec = jnp.full((16,), 0, jnp.int32) + cid
    def rm(j, c):
        for k in range(B // 16):
            v = row_v[j, 0, pl.ds(k * 16, 16)]
            row_v[j, 0, pl.ds(k * 16, 16)] = v + v + cvec
        return c
    lax.fori_loop(0, nb, rm, 0)

    for b in range(NBUF):
        pltpu.async_copy(y_hbm.at[row_v.at[b, 0]], bufs[b], sems[b])
    def outer(t, c):
        for b in range(NBUF):
            j = t * NBUF + b
            pltpu.make_async_copy(y_hbm.at[row_v.at[j, 0]], bufs[b],
                                  sems[b]).wait()
            pltpu.sync_copy(bufs[b], s_sh.at[col_v.at[j, 0]], add=True)
            @pl.when(j + NBUF < nb)
            def _():
                pltpu.async_copy(y_hbm.at[row_v.at[j + NBUF, 0]], bufs[b],
                                 sems[b])
        return c
    lax.fori_loop(0, BPT // NBUF, outer, 0)
    @pl.when(sid < XTRA)
    def _():
        pltpu.make_async_copy(y_hbm.at[row_v.at[BPT, 0]], bufs[0],
                              sems[0]).wait()
        pltpu.sync_copy(bufs[0], s_sh.at[col_v.at[BPT, 0]], add=True)
    plsc.subcore_barrier()
    pltpu.sync_copy(s_sh.at[pl.ds(sid * RPT, RPT)],
                    s_hbm.at[pl.ds(sid * RPT, RPT), pl.ds(cid * DH, DH)])


_gs_call = pl.kernel(
    _gs_body,
    out_type=jax.ShapeDtypeStruct((NPAD, D), jnp.float32),
    mesh=plsc.VectorSubcoreMesh(core_axis_name="c", subcore_axis_name="s",
                                num_cores=NC, num_subcores=NS),
    scratch_types=[
        pltpu.VMEM((MAXB, 1, B), jnp.int32),
        pltpu.VMEM((MAXB, 1, B), jnp.int32),
        pltpu.VMEM((B, DH), jnp.float32),
        pltpu.VMEM((B, DH), jnp.float32),
        pltpu.VMEM((B, DH), jnp.float32),
        pltpu.VMEM((B, DH), jnp.float32),
        pltpu.VMEM((B, DH), jnp.float32),
        pltpu.VMEM_SHARED((NPAD, DH), jnp.float32),
        pltpu.SemaphoreType.DMA,
        pltpu.SemaphoreType.DMA,
        pltpu.SemaphoreType.DMA,
        pltpu.SemaphoreType.DMA,
    ],
    compiler_params=_UNTILED,
)


def _sigmoid(x):
    return 1.0 / (1.0 + jnp.exp(-x))


def _y_body(x_ref, w_ref, wihT_ref, bi_ref, bh_ref, dd_ref, y_ref, wev_ref):
    @pl.when(pl.program_id(0) == 0)
    def _():
        g = jnp.dot(w_ref[...], wihT_ref[...],
                    preferred_element_type=jnp.float32)
        g = g + bi_ref[...] + bh_ref[...]
        i = g[:, :D]
        gg = g[:, 2 * D:3 * D]
        o = g[:, 3 * D:]
        c = _sigmoid(i) * jnp.tanh(gg)
        wev_ref[...] = _sigmoid(o) * jnp.tanh(c)
    dis = lax.rsqrt(dd_ref[...] + 1.0)
    y_ref[...] = jnp.dot(x_ref[...], wev_ref[...],
                         preferred_element_type=jnp.float32) * dis


def _y_call(X, W, W_ihT, bi, bh, dd):
    return pl.pallas_call(
        _y_body,
        grid=(N // RB,),
        in_specs=[
            pl.BlockSpec((RB, D), lambda i: (i, 0)),
            pl.BlockSpec((D, D), lambda i: (0, 0)),
            pl.BlockSpec((D, 4 * D), lambda i: (0, 0)),
            pl.BlockSpec((1, 4 * D), lambda i: (0, 0)),
            pl.BlockSpec((1, 4 * D), lambda i: (0, 0)),
            pl.BlockSpec((RB, 1), lambda i: (i, 0)),
        ],
        out_specs=pl.BlockSpec((RB, D), lambda i: (i, 0)),
        out_shape=jax.ShapeDtypeStruct((N, D), jnp.float32),
        scratch_shapes=[pltpu.VMEM((D, D), jnp.float32)],
    )(X, W, W_ihT, bi, bh, dd)


def _out_body(s_ref, y_ref, dd_ref, o_ref):
    dis = lax.rsqrt(dd_ref[...] + 1.0)
    o_ref[...] = (s_ref[...] + y_ref[...]) * dis


def _out_call(s, y, dd):
    return pl.pallas_call(
        _out_body,
        grid=(N // RB,),
        in_specs=[
            pl.BlockSpec((RB, D), lambda i: (i, 0)),
            pl.BlockSpec((RB, D), lambda i: (i, 0)),
            pl.BlockSpec((RB, 1), lambda i: (i, 0)),
        ],
        out_specs=pl.BlockSpec((RB, D), lambda i: (i, 0)),
        out_shape=jax.ShapeDtypeStruct((N, D), jnp.float32),
    )(s, y, dd)


def kernel(X, edge_index, W, W_ih, W_hh, b_ih, b_hh):
    ei3 = edge_index.reshape(2, NBT, B).transpose(1, 0, 2)

    d0, d1 = _deg_call(ei3)
    dd = (d0 + d1).reshape(NPAD, 1)
    Y = _y_call(X, W, W_ih.T, b_ih.reshape(1, 4 * D), b_hh.reshape(1, 4 * D),
                dd)
    y_view = Y.reshape(2 * N, DH)
    S = _gs_call(y_view, ei3)
    return _out_call(S, Y, dd)
